# baseline (speedup 1.0000x reference)
import functools

import jax
import jax.numpy as jnp
from jax import lax
from jax.experimental import pallas as pl
from jax.experimental.pallas import tpu as pltpu
from jax.experimental.pallas import tpu_sc as plsc

_NC = 2
_NS = 16
_NW = _NC * _NS
_CHUNK = 128


def _sc_aggregate(x, edges, n_pad, d):
    e = edges.shape[1]
    nfull = e // (_NW * _CHUNK)
    nextra = (e - _NW * nfull * _CHUNK) // _CHUNK
    ebase_extra = _NW * nfull * _CHUNK
    hch = nfull // 2
    rows_per_tile = n_pad // _NS
    copies_per_tile = rows_per_tile // _CHUNK
    mesh = plsc.VectorSubcoreMesh(core_axis_name="c", subcore_axis_name="s")

    @functools.partial(
        pl.kernel,
        out_type=jax.ShapeDtypeStruct((_NC, n_pad, d), jnp.float32),
        mesh=mesh,
        scratch_types=[
            pltpu.VMEM((2, hch * _CHUNK), jnp.int32),
            pltpu.VMEM((2, _CHUNK), jnp.int32),
            pltpu.VMEM((2, _CHUNK, d), jnp.float32),
            pltpu.VMEM_SHARED((n_pad, d), jnp.float32),
            pltpu.SemaphoreType.DMA,
            pltpu.SemaphoreType.DMA,
            pltpu.SemaphoreType.DMA,
            pltpu.SemaphoreType.DMA,
        ],
    )
    def agg_kernel(x_hbm, edge_hbm, out_hbm, idx_v, ext_v, buf_v, acc_sh,
                   gsem0, gsem1, ssem0, ssem1):
        c = lax.axis_index("c")
        s = lax.axis_index("s")
        wid = s * _NC + c
        ebase = wid * nfull * _CHUNK

        ext_wid = jnp.minimum(wid, nextra - 1)
        ext_cp = pltpu.async_copy(
            edge_hbm.at[:, pl.ds(ebase_extra + ext_wid * _CHUNK, _CHUNK)],
            ext_v, gsem1)

        zeros16 = jnp.zeros((16,), jnp.float32)
        lanes = d // 16

        def zero_body(r, carry):
            for k in range(lanes):
                buf_v[0, r, pl.ds(k * 16, 16)] = zeros16
            return carry

        lax.fori_loop(0, _CHUNK, zero_body, 0)
        row0 = s * rows_per_tile
        for r in range(copies_per_tile):
            pltpu.sync_copy(buf_v.at[0], acc_sh.at[pl.ds(row0 + r * _CHUNK,
                                                         _CHUNK)])
        ext_cp.wait()
        plsc.subcore_barrier()

        def sidx(j):
            return idx_v.at[0, pl.ds(j * _CHUNK, _CHUNK)]

        def didx(j):
            return idx_v.at[1, pl.ds(j * _CHUNK, _CHUNK)]

        for h in range(2):
            pltpu.sync_copy(
                edge_hbm.at[:, pl.ds(ebase + h * hch * _CHUNK,
                                     hch * _CHUNK)], idx_v)
            pltpu.async_copy(x_hbm.at[sidx(0)], buf_v.at[0], gsem0)
            pltpu.async_copy(x_hbm.at[sidx(1)], buf_v.at[1], gsem1)

            def pair_body(i, carry):
                j0 = 2 * i
                pltpu.make_async_copy(x_hbm.at[sidx(j0)], buf_v.at[0],
                                      gsem0).wait()
                pltpu.async_copy(buf_v.at[0], acc_sh.at[didx(j0)], ssem0,
                                 add=True)
                pltpu.make_async_copy(x_hbm.at[sidx(j0 + 1)], buf_v.at[1],
                                      gsem1).wait()
                pltpu.async_copy(buf_v.at[1], acc_sh.at[didx(j0 + 1)],
                                 ssem1, add=True)
                pltpu.make_async_copy(buf_v.at[0], acc_sh.at[didx(j0)],
                                      ssem0).wait()
                pltpu.async_copy(x_hbm.at[sidx(j0 + 2)], buf_v.at[0], gsem0)
                pltpu.make_async_copy(buf_v.at[1], acc_sh.at[didx(j0 + 1)],
                                      ssem1).wait()

                @pl.when(j0 + 3 < hch)
                def _():
                    pltpu.async_copy(x_hbm.at[sidx(j0 + 3)], buf_v.at[1],
                                     gsem1)

                return carry

            lax.fori_loop(0, hch // 2 - 1, pair_body, 0)
            j0 = 2 * (hch // 2 - 1)
            pltpu.make_async_copy(x_hbm.at[sidx(j0)], buf_v.at[0],
                                  gsem0).wait()
            pltpu.sync_copy(buf_v.at[0], acc_sh.at[didx(j0)], add=True)
            pltpu.make_async_copy(x_hbm.at[sidx(j0 + 1)], buf_v.at[1],
                                  gsem1).wait()
            pltpu.sync_copy(buf_v.at[1], acc_sh.at[didx(j0 + 1)], add=True)

            if hch % 2:
                j1 = hch - 1
                pltpu.async_copy(x_hbm.at[sidx(j1)], buf_v.at[0],
                                 gsem0).wait()
                pltpu.sync_copy(buf_v.at[0], acc_sh.at[didx(j1)], add=True)

        @pl.when(wid < nextra)
        def _():
            pltpu.async_copy(x_hbm.at[ext_v.at[0]], buf_v.at[0],
                             gsem0).wait()
            pltpu.sync_copy(buf_v.at[0], acc_sh.at[ext_v.at[1]], add=True)

        plsc.subcore_barrier()

        pltpu.sync_copy(acc_sh.at[pl.ds(row0, rows_per_tile)],
                        out_hbm.at[c, pl.ds(row0, rows_per_tile)])

    return agg_kernel(x, edges)


def _tc_project(parts, w, n, d):
    br = 5000
    grid = n // br

    def body(p_ref, w_ref, o_ref):
        agg = p_ref[0] + p_ref[1]
        o_ref[...] = jnp.maximum(
            jnp.dot(agg, w_ref[...], preferred_element_type=jnp.float32), 0.0)

    return pl.pallas_call(
        body,
        grid=(grid,),
        in_specs=[
            pl.BlockSpec((2, br, d), lambda i: (0, i, 0)),
            pl.BlockSpec((d, d), lambda i: (0, 0)),
        ],
        out_specs=pl.BlockSpec((br, d), lambda i: (i, 0)),
        out_shape=jax.ShapeDtypeStruct((n, d), jnp.float32),
    )(parts, w)


def _tc_copy_xh(x, h1, n, d):
    br = 5000
    sec = n // br

    def body(x_ref, h1_ref, o_ref):
        i = pl.program_id(0)

        @pl.when(i < sec)
        def _():
            o_ref[...] = x_ref[...]

        @pl.when(i >= sec)
        def _():
            o_ref[...] = h1_ref[...]

    clamp = lambda lo, hi: (lambda i: (jnp.clip(i - lo, 0, hi), 0))
    return pl.pallas_call(
        body,
        grid=(2 * sec,),
        in_specs=[
            pl.BlockSpec((br, d), clamp(0, sec - 1)),
            pl.BlockSpec((br, d), clamp(sec, sec - 1)),
        ],
        out_specs=pl.BlockSpec((br, d), lambda i: (i, 0)),
        out_shape=jax.ShapeDtypeStruct((3 * n, d), jnp.float32),
    )(x, h1)


def _tc_project_h2(out_xh, parts, w, n, d):
    br = 5000
    sec = n // br

    def body(o_in_ref, p_ref, w_ref, o_ref):
        agg = p_ref[0] + p_ref[1]
        o_ref[...] = jnp.maximum(
            jnp.dot(agg, w_ref[...], preferred_element_type=jnp.float32),
            0.0)

    return pl.pallas_call(
        body,
        grid=(sec,),
        in_specs=[
            pl.BlockSpec((8, d), lambda i: (0, 0)),
            pl.BlockSpec((2, br, d), lambda i: (0, i, 0)),
            pl.BlockSpec((d, d), lambda i: (0, 0)),
        ],
        out_specs=pl.BlockSpec((br, d), lambda i: (i + 2 * sec, 0)),
        out_shape=jax.ShapeDtypeStruct((3 * n, d), jnp.float32),
        input_output_aliases={0: 0},
    )(out_xh, parts, w)


def kernel(inputs, edge_index, W1, W2):
    n, d = inputs.shape
    n_pad = ((n + _NS * _CHUNK - 1) // (_NS * _CHUNK)) * (_NS * _CHUNK)

    p1 = _sc_aggregate(inputs, edge_index, n_pad, d)
    h1 = _tc_project(p1, W1, n, d)
    p2 = _sc_aggregate(h1, edge_index, n_pad, d)
    out_xh = _tc_copy_xh(inputs, h1, n, d)
    return _tc_project_h2(out_xh, p2, W2, n, d)


# speedup vs baseline: 1.2854x; 1.2854x over previous
import functools

import jax
import jax.numpy as jnp
from jax import lax
from jax.experimental import pallas as pl
from jax.experimental.pallas import tpu as pltpu
from jax.experimental.pallas import tpu_sc as plsc

_NC = 2
_NS = 16
_NW = _NC * _NS
_CHUNK = 128


def _sc_aggregate(x, edges, n_pad, d):
    e = edges.shape[1]
    nfull = e // (_NW * _CHUNK)
    nextra = (e - _NW * nfull * _CHUNK) // _CHUNK
    ebase_extra = _NW * nfull * _CHUNK
    hch = nfull // 2
    rows_per_tile = n_pad // _NS
    copies_per_tile = rows_per_tile // _CHUNK
    mesh = plsc.VectorSubcoreMesh(core_axis_name="c", subcore_axis_name="s")

    @functools.partial(
        pl.kernel,
        out_type=jax.ShapeDtypeStruct((_NC, n_pad, d), jnp.float32),
        mesh=mesh,
        scratch_types=[
            pltpu.VMEM((2, hch * _CHUNK), jnp.int32),
            pltpu.VMEM((2, _CHUNK), jnp.int32),
            pltpu.VMEM((2, _CHUNK, d), jnp.float32),
            pltpu.VMEM_SHARED((n_pad, d), jnp.float32),
            pltpu.SemaphoreType.DMA,
            pltpu.SemaphoreType.DMA,
        ],
    )
    def agg_kernel(x_hbm, edge_hbm, out_hbm, idx_v, ext_v, buf_v, acc_sh,
                   gsem0, gsem1):
        c = lax.axis_index("c")
        s = lax.axis_index("s")
        wid = s * _NC + c
        ebase = wid * nfull * _CHUNK

        idx0_cp = pltpu.async_copy(
            edge_hbm.at[:, pl.ds(ebase, hch * _CHUNK)], idx_v, gsem0)
        ext_wid = jnp.minimum(wid, nextra - 1)
        ext_cp = pltpu.async_copy(
            edge_hbm.at[:, pl.ds(ebase_extra + ext_wid * _CHUNK, _CHUNK)],
            ext_v, gsem1)

        zeros16 = jnp.zeros((16,), jnp.float32)
        lanes = d // 16

        def zero_body(r, carry):
            for k in range(lanes):
                buf_v[0, r, pl.ds(k * 16, 16)] = zeros16
            return carry

        lax.fori_loop(0, _CHUNK, zero_body, 0)
        row0 = s * rows_per_tile
        for r in range(copies_per_tile):
            pltpu.sync_copy(buf_v.at[0], acc_sh.at[pl.ds(row0 + r * _CHUNK,
                                                         _CHUNK)])
        ext_cp.wait()
        idx0_cp.wait()

        def sidx(j):
            return idx_v.at[0, pl.ds(j * _CHUNK, _CHUNK)]

        def didx(j):
            return idx_v.at[1, pl.ds(j * _CHUNK, _CHUNK)]

        pltpu.async_copy(x_hbm.at[sidx(0)], buf_v.at[0], gsem0)
        pltpu.async_copy(x_hbm.at[sidx(1)], buf_v.at[1], gsem1)
        plsc.subcore_barrier()

        for h in range(2):
            if h:
                pltpu.sync_copy(
                    edge_hbm.at[:, pl.ds(ebase + h * hch * _CHUNK,
                                         hch * _CHUNK)], idx_v)
                pltpu.async_copy(x_hbm.at[sidx(0)], buf_v.at[0], gsem0)
                pltpu.async_copy(x_hbm.at[sidx(1)], buf_v.at[1], gsem1)

            def pair_body(i, carry):
                j0 = 2 * i
                pltpu.make_async_copy(x_hbm.at[sidx(j0)], buf_v.at[0],
                                      gsem0).wait()
                pltpu.sync_copy(buf_v.at[0], acc_sh.at[didx(j0)], add=True)
                pltpu.async_copy(x_hbm.at[sidx(j0 + 2)], buf_v.at[0], gsem0)
                pltpu.make_async_copy(x_hbm.at[sidx(j0 + 1)], buf_v.at[1],
                                      gsem1).wait()
                pltpu.sync_copy(buf_v.at[1], acc_sh.at[didx(j0 + 1)],
                                add=True)

                @pl.when(j0 + 3 < hch)
                def _():
                    pltpu.async_copy(x_hbm.at[sidx(j0 + 3)], buf_v.at[1],
                                     gsem1)

                return carry

            lax.fori_loop(0, hch // 2 - 1, pair_body, 0)
            j0 = 2 * (hch // 2 - 1)
            pltpu.make_async_copy(x_hbm.at[sidx(j0)], buf_v.at[0],
                                  gsem0).wait()
            pltpu.sync_copy(buf_v.at[0], acc_sh.at[didx(j0)], add=True)
            pltpu.make_async_copy(x_hbm.at[sidx(j0 + 1)], buf_v.at[1],
                                  gsem1).wait()
            pltpu.sync_copy(buf_v.at[1], acc_sh.at[didx(j0 + 1)], add=True)

            if hch % 2:
                j1 = hch - 1
                pltpu.async_copy(x_hbm.at[sidx(j1)], buf_v.at[0],
                                 gsem0).wait()
                pltpu.sync_copy(buf_v.at[0], acc_sh.at[didx(j1)], add=True)

        @pl.when(wid < nextra)
        def _():
            pltpu.async_copy(x_hbm.at[ext_v.at[0]], buf_v.at[0],
                             gsem0).wait()
            pltpu.sync_copy(buf_v.at[0], acc_sh.at[ext_v.at[1]], add=True)

        plsc.subcore_barrier()

        pltpu.sync_copy(acc_sh.at[pl.ds(row0, rows_per_tile)],
                        out_hbm.at[c, pl.ds(row0, rows_per_tile)])

    return agg_kernel(x, edges)


def _tc_project(parts, w, n, d):
    br = 5000
    grid = n // br

    def body(p_ref, w_ref, o_ref):
        agg = p_ref[0] + p_ref[1]
        o_ref[...] = jnp.maximum(
            jnp.dot(agg, w_ref[...], preferred_element_type=jnp.float32), 0.0)

    return pl.pallas_call(
        body,
        grid=(grid,),
        in_specs=[
            pl.BlockSpec((2, br, d), lambda i: (0, i, 0)),
            pl.BlockSpec((d, d), lambda i: (0, 0)),
        ],
        out_specs=pl.BlockSpec((br, d), lambda i: (i, 0)),
        out_shape=jax.ShapeDtypeStruct((n, d), jnp.float32),
    )(parts, w)


def _tc_copy_xh(x, h1, n, d):
    br = 5000
    sec = n // br

    def body(x_ref, h1_ref, o_ref):
        i = pl.program_id(0)

        @pl.when(i < sec)
        def _():
            o_ref[...] = x_ref[...]

        @pl.when(i >= sec)
        def _():
            o_ref[...] = h1_ref[...]

    clamp = lambda lo, hi: (lambda i: (jnp.clip(i - lo, 0, hi), 0))
    return pl.pallas_call(
        body,
        grid=(2 * sec,),
        in_specs=[
            pl.BlockSpec((br, d), clamp(0, sec - 1)),
            pl.BlockSpec((br, d), clamp(sec, sec - 1)),
        ],
        out_specs=pl.BlockSpec((br, d), lambda i: (i, 0)),
        out_shape=jax.ShapeDtypeStruct((3 * n, d), jnp.float32),
    )(x, h1)


def _tc_project_h2(out_xh, parts, w, n, d):
    br = 5000
    sec = n // br

    def body(o_in_ref, p_ref, w_ref, o_ref):
        agg = p_ref[0] + p_ref[1]
        o_ref[...] = jnp.maximum(
            jnp.dot(agg, w_ref[...], preferred_element_type=jnp.float32),
            0.0)

    return pl.pallas_call(
        body,
        grid=(sec,),
        in_specs=[
            pl.BlockSpec((8, d), lambda i: (0, 0)),
            pl.BlockSpec((2, br, d), lambda i: (0, i, 0)),
            pl.BlockSpec((d, d), lambda i: (0, 0)),
        ],
        out_specs=pl.BlockSpec((br, d), lambda i: (i + 2 * sec, 0)),
        out_shape=jax.ShapeDtypeStruct((3 * n, d), jnp.float32),
        input_output_aliases={0: 0},
    )(out_xh, parts, w)


def kernel(inputs, edge_index, W1, W2):
    n, d = inputs.shape
    n_pad = ((n + _NS * _CHUNK - 1) // (_NS * _CHUNK)) * (_NS * _CHUNK)

    p1 = _sc_aggregate(inputs, edge_index, n_pad, d)
    h1 = _tc_project(p1, W1, n, d)
    p2 = _sc_aggregate(h1, edge_index, n_pad, d)
    out_xh = _tc_copy_xh(inputs, h1, n, d)
    return _tc_project_h2(out_xh, p2, W2, n, d)
